# Initial kernel scaffold; baseline (speedup 1.0000x reference)
import jax
import jax.numpy as jnp
from jax.experimental import pallas as pl


def kernel(seq, adj_indices, adj_values, weight, W_fts, a1, b1, a2, b2, bias):
    raise NotImplementedError("write your pallas kernel here")


# speedup vs baseline: 10.3850x; 10.3850x over previous
import jax
import jax.numpy as jnp
from jax import lax
from jax.experimental import pallas as pl
from jax.experimental.pallas import tpu as pltpu
from jax.experimental.pallas import tpu_sc as plsc

N = 10000
E = 160000
F = 256
FH = 128
N_PAD = 10240
E_PAD = 163840
G = 128
NROWS2D = E_PAD // G
NCORES = 2
NSUB = 16
CHUNKS_A = NROWS2D // (NCORES * NSUB)
CHUNKS_B = NROWS2D // NSUB
GB = 16
RPT = N_PAD // NSUB
FIN = 128
BM = 1024


def _tc_body(x_ref, w_ref, a12_ref, b12_ref, fts2_ref, f12_ref):
    y = jnp.dot(x_ref[...], w_ref[...], preferred_element_type=jnp.float32)
    fts2_ref[0] = y[:, :FH]
    fts2_ref[1] = y[:, FH:]
    f12_ref[...] = jnp.dot(y, a12_ref[...],
                           preferred_element_type=jnp.float32) + b12_ref[...]


def _tc_stage(seq_p, W_fts, a12, b12):
    nblk = N_PAD // BM
    return pl.pallas_call(
        _tc_body,
        grid=(nblk,),
        in_specs=[
            pl.BlockSpec((BM, F), lambda i: (i, 0)),
            pl.BlockSpec((F, F), lambda i: (0, 0)),
            pl.BlockSpec((F, 2), lambda i: (0, 0)),
            pl.BlockSpec((1, 2), lambda i: (0, 0)),
        ],
        out_specs=[
            pl.BlockSpec((2, BM, FH), lambda i: (0, i, 0)),
            pl.BlockSpec((BM, 2), lambda i: (i, 0)),
        ],
        out_shape=[
            jax.ShapeDtypeStruct((2, N_PAD, FH), jnp.float32),
            jax.ShapeDtypeStruct((N_PAD, 2), jnp.float32),
        ],
    )(seq_p, W_fts, a12, b12)


def _make_sc_edge():
    mesh = plsc.VectorSubcoreMesh(core_axis_name="c", subcore_axis_name="s",
                                  num_cores=NCORES, num_subcores=NSUB)

    def body(row_hbm, col_hbm, av_hbm, w_hbm, f1_hbm, f2_hbm, zeros_hbm,
             e_hbm, rs_hbm,
             row_v, col_v, av_v, w_v, f1_v, f2_v, e_v, rs_sh):
        c = lax.axis_index("c")
        s = lax.axis_index("s")
        wid = s * NCORES + c
        base = wid * CHUNKS_A

        pltpu.sync_copy(row_hbm.at[pl.ds(base, CHUNKS_A)], row_v)
        pltpu.sync_copy(col_hbm.at[pl.ds(base, CHUNKS_A)], col_v)
        pltpu.sync_copy(av_hbm.at[pl.ds(base, CHUNKS_A)], av_v)
        pltpu.sync_copy(w_hbm.at[pl.ds(base, CHUNKS_A)], w_v)
        pltpu.sync_copy(f1_hbm, f1_v)
        pltpu.sync_copy(f2_hbm, f2_v)

        @pl.when(s == 0)
        def _():
            pltpu.sync_copy(zeros_hbm, rs_sh)

        plsc.subcore_barrier()

        def chunk(j, _):
            for k in range(G // 16):
                dk = pl.ds(k * 16, 16)
                ir = row_v[j, dk]
                ic = col_v[j, dk]
                f1g = plsc.load_gather(f1_v, [ir])
                f2g = plsc.load_gather(f2_v, [ic])
                av = av_v[j, dk]
                lg = av * f1g + av * f2g
                val = jnp.where(lg > 0.0, lg, 0.2 * lg) * w_v[j, dk]
                e_v[j, dk] = jnp.exp(val)
            pltpu.sync_copy(e_v.at[j], rs_sh.at[row_v.at[j]], add=True)
            return 0

        lax.fori_loop(0, CHUNKS_A, chunk, 0)
        pltpu.sync_copy(e_v, e_hbm.at[pl.ds(base, CHUNKS_A)])
        plsc.subcore_barrier()

        @pl.when(s == 0)
        def _():
            pltpu.sync_copy(rs_sh, rs_hbm.at[c])

    return pl.kernel(
        body,
        out_type=[
            jax.ShapeDtypeStruct((NROWS2D, G), jnp.float32),
            jax.ShapeDtypeStruct((NCORES, N_PAD), jnp.float32),
        ],
        mesh=mesh,
        compiler_params=pltpu.CompilerParams(needs_layout_passes=False),
        scratch_types=[
            pltpu.VMEM((CHUNKS_A, G), jnp.int32),
            pltpu.VMEM((CHUNKS_A, G), jnp.int32),
            pltpu.VMEM((CHUNKS_A, G), jnp.float32),
            pltpu.VMEM((CHUNKS_A, G), jnp.float32),
            pltpu.VMEM((N_PAD,), jnp.float32),
            pltpu.VMEM((N_PAD,), jnp.float32),
            pltpu.VMEM((CHUNKS_A, G), jnp.float32),
            pltpu.VMEM_SHARED((N_PAD,), jnp.float32),
        ],
    )


def _make_sc_accum():
    mesh = plsc.VectorSubcoreMesh(core_axis_name="c", subcore_axis_name="s",
                                  num_cores=NCORES, num_subcores=NSUB)

    def body(row_hbm, col_hbm, e_hbm, rs0_hbm, rs1_hbm, fts_hbm, bias2_hbm,
             zeros_hbm, out_hbm,
             row_v, col_v, e_v, rows_a, rows_b, rsa_v, rsb_v, bias_v,
             acc_sh, msem, gsem, ssem):
        c = lax.axis_index("c")
        s = lax.axis_index("s")
        base = s * CHUNKS_B

        pltpu.sync_copy(zeros_hbm.at[pl.ds(s * RPT, RPT)],
                        acc_sh.at[pl.ds(s * RPT, RPT)])
        plsc.subcore_barrier()

        cN = c * N_PAD

        def stage_meta(gb):
            m0 = pltpu.async_copy(row_hbm.at[pl.ds(gb, GB)], row_v, msem)
            m1 = pltpu.async_copy(col_hbm.at[pl.ds(gb, GB)], col_v, msem)
            m2 = pltpu.async_copy(e_hbm.at[pl.ds(gb, GB)], e_v, msem)
            m0.wait()
            m1.wait()
            m2.wait()
            def mk_idx(j, _):
                for k in range(G // 16):
                    dk = pl.ds(k * 16, 16)
                    col_v[j, dk] = col_v[j, dk] + cN
                return 0
            lax.fori_loop(0, GB, mk_idx, 0)

        def scale(buf, j):
            jb = lax.broadcast(j, (16,))

            def srow(i, _):
                i0 = i * 4
                evs = [plsc.load_gather(e_v, [jb, lax.broadcast(i0 + u, (16,))])
                       for u in range(4)]
                for u in range(4):
                    for f in range(FH // 16):
                        df = pl.ds(f * 16, 16)
                        buf[i0 + u, df] = buf[i0 + u, df] * evs[u]
                return 0

            lax.fori_loop(0, G // 4, srow, 0)

        def group(g, _):
            stage_meta(base + g * GB)
            sg_a = pltpu.async_copy(fts_hbm.at[col_v.at[0]], rows_a, gsem)
            for t in range(GB // 2):
                j0 = 2 * t
                j1 = j0 + 1
                sg_b = pltpu.async_copy(fts_hbm.at[col_v.at[j1]], rows_b, gsem)
                sg_a.wait()
                scale(rows_a, j0)
                ss_a = pltpu.async_copy(rows_a, acc_sh.at[row_v.at[j0]], ssem,
                                        add=True)
                sg_b.wait()
                scale(rows_b, j1)
                ss_b = pltpu.async_copy(rows_b, acc_sh.at[row_v.at[j1]], ssem,
                                        add=True)
                ss_a.wait()
                if t < GB // 2 - 1:
                    sg_a = pltpu.async_copy(fts_hbm.at[col_v.at[j0 + 2]],
                                            rows_a, gsem)
                ss_b.wait()
            return 0

        lax.fori_loop(0, CHUNKS_B // GB, group, 0)
        plsc.subcore_barrier()

        pltpu.sync_copy(bias2_hbm.at[c], bias_v)
        for h in range(RPT // FIN):
            r0 = s * RPT + h * FIN
            pltpu.sync_copy(acc_sh.at[pl.ds(r0, FIN)], rows_a)
            pltpu.sync_copy(rs0_hbm.at[pl.ds(r0, FIN)], rsa_v)
            pltpu.sync_copy(rs1_hbm.at[pl.ds(r0, FIN)], rsb_v)

            def finrow(i, _):
                ib = lax.broadcast(i, (16,))
                rsv = (plsc.load_gather(rsa_v, [ib])
                       + plsc.load_gather(rsb_v, [ib]))
                d = jnp.where(rsv == 0.0, 1.0, rsv)
                inv = 1.0 / d
                for f in range(FH // 16):
                    df = pl.ds(f * 16, 16)
                    v = rows_a[i, df] * inv + bias_v[df]
                    rows_a[i, df] = jnp.maximum(v, 0.0)
                return 0

            lax.fori_loop(0, FIN, finrow, 0)
            pltpu.sync_copy(rows_a, out_hbm.at[c, pl.ds(r0, FIN)])

    return pl.kernel(
        body,
        out_type=jax.ShapeDtypeStruct((NCORES, N_PAD, FH), jnp.float32),
        mesh=mesh,
        compiler_params=pltpu.CompilerParams(needs_layout_passes=False),
        scratch_types=[
            pltpu.VMEM((GB, G), jnp.int32),
            pltpu.VMEM((GB, G), jnp.int32),
            pltpu.VMEM((GB, G), jnp.float32),
            pltpu.VMEM((G, FH), jnp.float32),
            pltpu.VMEM((G, FH), jnp.float32),
            pltpu.VMEM((FIN,), jnp.float32),
            pltpu.VMEM((FIN,), jnp.float32),
            pltpu.VMEM((FH,), jnp.float32),
            pltpu.VMEM_SHARED((N_PAD, FH), jnp.float32),
            pltpu.SemaphoreType.DMA,
            pltpu.SemaphoreType.DMA,
            pltpu.SemaphoreType.DMA,
        ],
    )


_sc_edge = _make_sc_edge()
_sc_accum = _make_sc_accum()


@jax.jit
def kernel(seq, adj_indices, adj_values, weight, W_fts, a1, b1, a2, b2, bias):
    row = adj_indices[0].astype(jnp.int32)
    col = adj_indices[1].astype(jnp.int32)

    seq_p = jnp.pad(seq, ((0, N_PAD - N), (0, 0)))
    a12 = jnp.concatenate([a1, a2], axis=1)
    b12 = jnp.concatenate([b1, b2]).reshape(1, 2)

    fts2, f12 = _tc_stage(seq_p, W_fts, a12, b12)
    fts_flat = fts2.reshape(NCORES * N_PAD, FH)
    f1 = f12[:, 0]
    f2 = f12[:, 1]

    pad_e = E_PAD - E
    row_p = jnp.pad(row, (0, pad_e), constant_values=N_PAD - 1).reshape(NROWS2D, G)
    col_p = jnp.pad(col, (0, pad_e)).reshape(NROWS2D, G)
    av_p = jnp.pad(adj_values, (0, pad_e)).reshape(NROWS2D, G)
    w_p = jnp.pad(weight, (0, pad_e)).reshape(NROWS2D, G)

    zeros2d = jnp.zeros((N_PAD, FH), jnp.float32)
    zeros1d = jnp.zeros((N_PAD,), jnp.float32)

    e, rs = _sc_edge(row_p, col_p, av_p, w_p, f1, f2, zeros1d)

    bias2 = bias.reshape(NCORES, FH)
    out2 = _sc_accum(row_p, col_p, e, rs[0], rs[1], fts_flat, bias2, zeros2d)
    return jnp.concatenate([out2[0], out2[1]], axis=1)[:N]
